# Initial kernel scaffold; baseline (speedup 1.0000x reference)
import jax
import jax.numpy as jnp
from jax.experimental import pallas as pl


def kernel(features, w_fused, b_fused):
    raise NotImplementedError("write your pallas kernel here")


# speedup vs baseline: 1.4342x; 1.4342x over previous
import jax
import jax.numpy as jnp
from jax.experimental import pallas as pl
from jax.experimental.pallas import tpu as pltpu

_P = 4
_TB = 4096


def _head_kernel(x_ref, w_ref, b_ref, pi_ref, vf_ref):
    acc = jnp.dot(x_ref[...], w_ref[...], preferred_element_type=jnp.float32)
    out = acc + b_ref[...].astype(jnp.float32)
    pi_ref[...] = out[:, :_P].astype(pi_ref.dtype)
    vf_ref[...] = out[:, _P:].astype(vf_ref.dtype)


def kernel(features, w_fused, b_fused):
    B, F = features.shape
    OUT = w_fused.shape[1]
    V = OUT - _P
    out_dtype = jnp.result_type(features.dtype, w_fused.dtype)
    b_fused = b_fused.reshape(1, OUT)

    tb = min(_TB, B)
    grid = (pl.cdiv(B, tb),)

    pi, vf = pl.pallas_call(
        _head_kernel,
        grid=grid,
        in_specs=[
            pl.BlockSpec((tb, F), lambda i: (i, 0)),
            pl.BlockSpec((F, OUT), lambda i: (0, 0)),
            pl.BlockSpec((1, OUT), lambda i: (0, 0)),
        ],
        out_specs=[
            pl.BlockSpec((tb, _P), lambda i: (i, 0)),
            pl.BlockSpec((tb, V), lambda i: (i, 0)),
        ],
        out_shape=[
            jax.ShapeDtypeStruct((B, _P), out_dtype),
            jax.ShapeDtypeStruct((B, V), out_dtype),
        ],
        compiler_params=pltpu.CompilerParams(
            dimension_semantics=("parallel",),
            vmem_limit_bytes=64 << 20,
        ),
    )(features, w_fused, b_fused)
    return pi, vf


# speedup vs baseline: 1.4587x; 1.0171x over previous
import jax
import jax.numpy as jnp
from jax.experimental import pallas as pl
from jax.experimental.pallas import tpu as pltpu

_P = 4
_TB = 8192


def _head_kernel(x_ref, w_ref, b_ref, pi_ref, vf_ref):
    acc = jnp.dot(x_ref[...], w_ref[...], preferred_element_type=jnp.float32)
    out = acc + b_ref[...].astype(jnp.float32)
    pi_ref[...] = out[:, :_P].astype(pi_ref.dtype)
    vf_ref[...] = out[:, _P:].astype(vf_ref.dtype)


def kernel(features, w_fused, b_fused):
    B, F = features.shape
    OUT = w_fused.shape[1]
    V = OUT - _P
    out_dtype = jnp.result_type(features.dtype, w_fused.dtype)
    b_fused = b_fused.reshape(1, OUT)

    tb = min(_TB, B)
    grid = (pl.cdiv(B, tb),)

    pi, vf = pl.pallas_call(
        _head_kernel,
        grid=grid,
        in_specs=[
            pl.BlockSpec((tb, F), lambda i: (i, 0)),
            pl.BlockSpec((F, OUT), lambda i: (0, 0)),
            pl.BlockSpec((1, OUT), lambda i: (0, 0)),
        ],
        out_specs=[
            pl.BlockSpec((tb, _P), lambda i: (i, 0)),
            pl.BlockSpec((tb, V), lambda i: (i, 0)),
        ],
        out_shape=[
            jax.ShapeDtypeStruct((B, _P), out_dtype),
            jax.ShapeDtypeStruct((B, V), out_dtype),
        ],
        compiler_params=pltpu.CompilerParams(
            dimension_semantics=("parallel",),
            vmem_limit_bytes=64 << 20,
        ),
    )(features, w_fused, b_fused)
    return pi, vf


# speedup vs baseline: 1.4606x; 1.0013x over previous
import jax
import jax.numpy as jnp
from jax.experimental import pallas as pl
from jax.experimental.pallas import tpu as pltpu

_P = 4
_TB = 4096


def _head_kernel(x0_ref, x1_ref, w_ref, b_ref, pi_ref, vf_ref):
    b = b_ref[...].astype(jnp.float32)
    tb = x0_ref.shape[0]
    out0 = jnp.dot(x0_ref[...], w_ref[...],
                   preferred_element_type=jnp.float32) + b
    pi_ref[:tb, :] = out0[:, :_P].astype(pi_ref.dtype)
    vf_ref[:tb, :] = out0[:, _P:].astype(vf_ref.dtype)
    out1 = jnp.dot(x1_ref[...], w_ref[...],
                   preferred_element_type=jnp.float32) + b
    pi_ref[tb:, :] = out1[:, :_P].astype(pi_ref.dtype)
    vf_ref[tb:, :] = out1[:, _P:].astype(vf_ref.dtype)


def kernel(features, w_fused, b_fused):
    B, F = features.shape
    OUT = w_fused.shape[1]
    V = OUT - _P
    out_dtype = jnp.result_type(features.dtype, w_fused.dtype)
    b_fused = b_fused.reshape(1, OUT)

    tb = max(min(_TB, B // 2), 1)
    n = pl.cdiv(B, 2 * tb)
    grid = (n,)

    pi, vf = pl.pallas_call(
        _head_kernel,
        grid=grid,
        in_specs=[
            pl.BlockSpec((tb, F), lambda i: (2 * i, 0)),
            pl.BlockSpec((tb, F), lambda i: (2 * i + 1, 0)),
            pl.BlockSpec((F, OUT), lambda i: (0, 0)),
            pl.BlockSpec((1, OUT), lambda i: (0, 0)),
        ],
        out_specs=[
            pl.BlockSpec((2 * tb, _P), lambda i: (i, 0)),
            pl.BlockSpec((2 * tb, V), lambda i: (i, 0)),
        ],
        out_shape=[
            jax.ShapeDtypeStruct((B, _P), out_dtype),
            jax.ShapeDtypeStruct((B, V), out_dtype),
        ],
        compiler_params=pltpu.CompilerParams(
            dimension_semantics=("parallel",),
            vmem_limit_bytes=64 << 20,
        ),
    )(features, features, w_fused, b_fused)
    return pi, vf
